# Initial kernel scaffold; baseline (speedup 1.0000x reference)
import jax
import jax.numpy as jnp
from jax.experimental import pallas as pl


def kernel(img):
    raise NotImplementedError("write your pallas kernel here")


# speedup vs baseline: 8584.9380x; 8584.9380x over previous
import numpy as np
import jax
import jax.numpy as jnp
from jax.experimental import pallas as pl

_H, _W, _C = 512, 512, 3
_SIGMA = 0.4
_RADIUS = 2
_MAX_DELTA = 1
_WC = _W * _C


def _blur_taps() -> np.ndarray:
    x = np.arange(-_RADIUS, _RADIUS + 1)
    k = np.exp(-0.5 * (x / _SIGMA) ** 2)
    return (k / k.sum()).astype(np.float32)


_K = _blur_taps()

_MASKS_CACHE = None


def _displacement_masks():
    global _MASKS_CACHE
    if _MASKS_CACHE is None:
        n = (_H - 2 * _MAX_DELTA) * (_W - 2 * _MAX_DELTA)
        with jax.ensure_compile_time_eval():
            dxy = jax.random.randint(
                jax.random.key(1), (n, 2), -_MAX_DELTA, _MAX_DELTA,
                dtype=jnp.int32,
            )
        d = np.asarray(dxy).reshape(_H - 2, _W - 2, 2)
        a = np.zeros((_H, _W), np.float32)
        b = np.zeros((_H, _W), np.float32)
        a[2:, 2:] = -d[::-1, ::-1, 1].astype(np.float32)
        b[2:, 2:] = -d[::-1, ::-1, 0].astype(np.float32)
        a = np.repeat(a, _C, axis=1)
        b = np.repeat(b, _C, axis=1)
        _MASKS_CACHE = (a, b)
    return _MASKS_CACHE


def _shift_rows(x, d):
    if d < 0:
        return jnp.concatenate([jnp.broadcast_to(x[:1], (-d,) + x.shape[1:]),
                                x[:d]], axis=0)
    if d > 0:
        return jnp.concatenate([x[d:],
                                jnp.broadcast_to(x[-1:], (d,) + x.shape[1:])],
                               axis=0)
    return x


def _shift_pixels(x, d):
    L = d * _C
    if d < 0:
        edge = x[:, :_C]
        reps = [edge] * (-d) + [x[:, :L]]
        return jnp.concatenate(reps, axis=1)
    if d > 0:
        edge = x[:, -_C:]
        reps = [x[:, L:]] + [edge] * d
        return jnp.concatenate(reps, axis=1)
    return x


def _blur2d(x):
    acc = _K[_RADIUS] * x
    for r in range(1, _RADIUS + 1):
        acc = acc + _K[_RADIUS - r] * (_shift_rows(x, -r) + _shift_rows(x, r))
    x = acc
    acc = _K[_RADIUS] * x
    for r in range(1, _RADIUS + 1):
        acc = acc + _K[_RADIUS - r] * (_shift_pixels(x, -r) + _shift_pixels(x, r))
    return acc


def _glass_blur_body(x_ref, a_ref, b_ref, o_ref):
    x = _blur2d(x_ref[...])
    a = a_ref[...]
    b = b_ref[...]
    x_up = _shift_rows(x, -1)
    x_lf = _shift_pixels(x, -1)
    x_ul = _shift_pixels(x_up, -1)
    g = ((1.0 - a) * ((1.0 - b) * x + b * x_lf)
         + a * ((1.0 - b) * x_up + b * x_ul))
    o_ref[...] = jnp.clip(_blur2d(g), 0.0, 1.0)


def kernel(img):
    a, b = _displacement_masks()
    flat = img.reshape(_H, _WC)
    out = pl.pallas_call(
        _glass_blur_body,
        out_shape=jax.ShapeDtypeStruct((_H, _WC), jnp.float32),
    )(flat, jnp.asarray(a), jnp.asarray(b))
    return out.reshape(_H, _W, _C)
